# Initial kernel scaffold; baseline (speedup 1.0000x reference)
import jax
import jax.numpy as jnp
from jax.experimental import pallas as pl


def kernel(x, edge_index, W, b, gamma, beta):
    raise NotImplementedError("write your pallas kernel here")


# speedup vs baseline: 27.0533x; 27.0533x over previous
import functools

import jax
import jax.numpy as jnp
from jax import lax
from jax.experimental import pallas as pl
from jax.experimental.pallas import tpu as pltpu
from jax.experimental.pallas import tpu_sc as plsc

N_NODES = 10000
DIM = 128
N_EDGES = 320000
BN_EPS = 1e-5

NC, NS = 2, 16
NW = NC * NS
NPAD = 10240
RPT = NPAD // NS
EPT = N_EDGES // NW
K = 125
NCH = EPT // K


def _sc_mesh():
    return plsc.VectorSubcoreMesh(
        core_axis_name="c", subcore_axis_name="s", num_cores=NC, num_subcores=NS
    )


@functools.partial(
    pl.kernel,
    out_type=jax.ShapeDtypeStruct((NC, NPAD), jnp.float32),
    mesh=_sc_mesh(),
    scratch_types=[
        pltpu.VMEM((NCH, K), jnp.int32),
        pltpu.VMEM((128,), jnp.float32),
        pltpu.VMEM_SHARED((NPAD,), jnp.float32),
    ],
)
def _deg_kernel(dst_hbm, zero_hbm, out_hbm, idx_v, ones_v, acc):
    cid = lax.axis_index("c")
    sid = lax.axis_index("s")
    wid = cid * NS + sid
    pltpu.sync_copy(dst_hbm.at[wid], idx_v)
    for i in range(8):
        ones_v[pl.ds(i * 16, 16)] = jnp.ones((16,), jnp.float32)
    pltpu.sync_copy(zero_hbm.at[pl.ds(sid * RPT, RPT)], acc.at[pl.ds(sid * RPT, RPT)])
    plsc.subcore_barrier()

    def body(j, carry):
        pltpu.sync_copy(ones_v.at[pl.ds(0, K)], acc.at[idx_v.at[j]], add=True)
        return carry

    lax.fori_loop(0, NCH, body, 0)
    plsc.subcore_barrier()
    pltpu.sync_copy(acc.at[pl.ds(sid * RPT, RPT)], out_hbm.at[cid, pl.ds(sid * RPT, RPT)])


@functools.partial(
    pl.kernel,
    out_type=jax.ShapeDtypeStruct((NC, NPAD, DIM), jnp.float32),
    mesh=_sc_mesh(),
    scratch_types=[
        pltpu.VMEM((NCH, K), jnp.int32),
        pltpu.VMEM((NCH, K), jnp.int32),
        pltpu.VMEM((K, DIM), jnp.float32),
        pltpu.SemaphoreType.DMA,
        pltpu.VMEM_SHARED((NPAD, DIM), jnp.float32),
    ],
)
def _agg_kernel(y_hbm, src_hbm, dst_hbm, zero_hbm, out_hbm, sidx, didx, buf, sem, acc):
    cid = lax.axis_index("c")
    sid = lax.axis_index("s")
    wid = cid * NS + sid
    pltpu.sync_copy(src_hbm.at[wid], sidx)
    pltpu.sync_copy(dst_hbm.at[wid], didx)
    pltpu.sync_copy(
        zero_hbm.at[pl.ds(sid * RPT, RPT)], acc.at[pl.ds(sid * RPT, RPT)]
    )
    plsc.subcore_barrier()

    def body(j, carry):
        pltpu.async_copy(y_hbm.at[sidx.at[j]], buf, sem).wait()
        pltpu.sync_copy(buf, acc.at[didx.at[j]], add=True)
        return carry

    lax.fori_loop(0, NCH, body, 0)
    plsc.subcore_barrier()
    pltpu.sync_copy(
        acc.at[pl.ds(sid * RPT, RPT)], out_hbm.at[cid, pl.ds(sid * RPT, RPT)]
    )


def _prep_body(x_ref, w_ref, degt_ref, y_ref):
    deg = degt_ref[:, 0:1] + degt_ref[:, 1:2] + 1.0
    dis = lax.rsqrt(deg)
    x_lin = jnp.dot(x_ref[...], w_ref[...], preferred_element_type=jnp.float32)
    y_ref[...] = x_lin * dis


def _post_body(aggp_ref, y_ref, degt_ref, x_ref, b_ref, g_ref, be_ref, o_ref):
    a = aggp_ref[0][: N_NODES, :] + aggp_ref[1][: N_NODES, :] + y_ref[: N_NODES, :]
    deg = degt_ref[: N_NODES, 0:1] + degt_ref[: N_NODES, 1:2] + 1.0
    dis = lax.rsqrt(deg)
    out = a * dis + b_ref[...]
    mean = jnp.mean(out, axis=0, keepdims=True)
    var = jnp.mean((out - mean) * (out - mean), axis=0, keepdims=True)
    bn = g_ref[...] * (out - mean) * lax.rsqrt(var + BN_EPS) + be_ref[...]
    o_ref[...] = jnp.maximum(bn, 0.0) + x_ref[...]


@jax.jit
def kernel(x, edge_index, W, b, gamma, beta):
    src = edge_index[0].astype(jnp.int32).reshape(NW, NCH, K)
    dst = edge_index[1].astype(jnp.int32).reshape(NW, NCH, K)
    x_pad = jnp.pad(x, ((0, NPAD - N_NODES), (0, 0)))
    zero_deg = jnp.zeros((NPAD,), jnp.float32)
    zero_rows = jnp.zeros((NPAD, DIM), jnp.float32)

    degp = _deg_kernel(dst, zero_deg)
    degt = degp.T

    y = pl.pallas_call(
        _prep_body,
        out_shape=jax.ShapeDtypeStruct((NPAD, DIM), jnp.float32),
    )(x_pad, W, degt)

    aggp = _agg_kernel(y, src, dst, zero_rows)

    out = pl.pallas_call(
        _post_body,
        out_shape=jax.ShapeDtypeStruct((N_NODES, DIM), jnp.float32),
    )(aggp, y, degt, x, b.reshape(1, DIM), gamma.reshape(1, DIM), beta.reshape(1, DIM))
    return out


# speedup vs baseline: 32.8611x; 1.2147x over previous
import functools

import jax
import jax.numpy as jnp
from jax import lax
from jax.experimental import pallas as pl
from jax.experimental.pallas import tpu as pltpu
from jax.experimental.pallas import tpu_sc as plsc

N_NODES = 10000
DIM = 128
N_EDGES = 320000
BN_EPS = 1e-5

NC, NS = 2, 16
NW = NC * NS
NPAD = 10240
RPT = NPAD // NS
EPT = N_EDGES // NW
K = 125
NCH = EPT // K
NHALF = NCH // 2


def _sc_mesh():
    return plsc.VectorSubcoreMesh(
        core_axis_name="c", subcore_axis_name="s", num_cores=NC, num_subcores=NS
    )


@functools.partial(
    pl.kernel,
    out_type=jax.ShapeDtypeStruct((NC, NPAD), jnp.float32),
    mesh=_sc_mesh(),
    scratch_types=[
        pltpu.VMEM((NCH, K), jnp.int32),
        pltpu.VMEM((128,), jnp.float32),
        pltpu.VMEM_SHARED((NPAD,), jnp.float32),
    ],
)
def _deg_kernel(dst_hbm, zero_hbm, out_hbm, idx_v, ones_v, acc):
    cid = lax.axis_index("c")
    sid = lax.axis_index("s")
    wid = cid * NS + sid
    pltpu.sync_copy(dst_hbm.at[wid], idx_v)
    for i in range(8):
        ones_v[pl.ds(i * 16, 16)] = jnp.ones((16,), jnp.float32)
    pltpu.sync_copy(zero_hbm.at[pl.ds(sid * RPT, RPT)], acc.at[pl.ds(sid * RPT, RPT)])
    plsc.subcore_barrier()

    def body(j, carry):
        pltpu.sync_copy(ones_v.at[pl.ds(0, K)], acc.at[idx_v.at[j]], add=True)
        return carry

    lax.fori_loop(0, NCH, body, 0)
    plsc.subcore_barrier()
    pltpu.sync_copy(acc.at[pl.ds(sid * RPT, RPT)], out_hbm.at[cid, pl.ds(sid * RPT, RPT)])


@functools.partial(
    pl.kernel,
    out_type=jax.ShapeDtypeStruct((NC, NPAD, DIM), jnp.float32),
    mesh=_sc_mesh(),
    scratch_types=[
        pltpu.VMEM((NHALF, K), jnp.int32),
        pltpu.VMEM((NHALF, K), jnp.int32),
        pltpu.VMEM((K, DIM), jnp.float32),
        pltpu.VMEM((K, DIM), jnp.float32),
        pltpu.SemaphoreType.DMA,
        pltpu.SemaphoreType.DMA,
        pltpu.VMEM_SHARED((NPAD, DIM), jnp.float32),
    ],
)
def _agg_kernel(
    y_hbm, src_hbm, dst_hbm, zero_hbm, out_hbm,
    sidx, didx, bufa, bufb, gsa, gsb, acc,
):
    cid = lax.axis_index("c")
    sid = lax.axis_index("s")
    wid = cid * NS + sid
    pltpu.sync_copy(
        zero_hbm.at[pl.ds(sid * RPT, RPT)], acc.at[pl.ds(sid * RPT, RPT)]
    )
    plsc.subcore_barrier()

    NT = NHALF // 2
    for h in range(2):
        pltpu.sync_copy(src_hbm.at[wid, pl.ds(h * NHALF, NHALF)], sidx)
        pltpu.sync_copy(dst_hbm.at[wid, pl.ds(h * NHALF, NHALF)], didx)
        pltpu.async_copy(y_hbm.at[sidx.at[0]], bufa, gsa)

        def body(t, carry):
            j0 = 2 * t
            j1 = j0 + 1
            pltpu.make_async_copy(y_hbm.at[sidx.at[j0]], bufa, gsa).wait()
            pltpu.async_copy(y_hbm.at[sidx.at[j1]], bufb, gsb)
            pltpu.sync_copy(bufa, acc.at[didx.at[j0]], add=True)
            pltpu.make_async_copy(y_hbm.at[sidx.at[j1]], bufb, gsb).wait()

            @pl.when(t + 1 < NT)
            def _prefetch():
                pltpu.async_copy(y_hbm.at[sidx.at[j0 + 2]], bufa, gsa)

            pltpu.sync_copy(bufb, acc.at[didx.at[j1]], add=True)
            return carry

        lax.fori_loop(0, NT, body, 0)
    plsc.subcore_barrier()
    pltpu.sync_copy(
        acc.at[pl.ds(sid * RPT, RPT)], out_hbm.at[cid, pl.ds(sid * RPT, RPT)]
    )


def _prep_body(x_ref, w_ref, degt_ref, y_ref):
    deg = degt_ref[:, 0:1] + degt_ref[:, 1:2] + 1.0
    dis = lax.rsqrt(deg)
    x_lin = jnp.dot(x_ref[...], w_ref[...], preferred_element_type=jnp.float32)
    y_ref[...] = x_lin * dis


def _post_body(aggp_ref, y_ref, degt_ref, x_ref, b_ref, g_ref, be_ref, o_ref):
    a = aggp_ref[0][: N_NODES, :] + aggp_ref[1][: N_NODES, :] + y_ref[: N_NODES, :]
    deg = degt_ref[: N_NODES, 0:1] + degt_ref[: N_NODES, 1:2] + 1.0
    dis = lax.rsqrt(deg)
    out = a * dis + b_ref[...]
    mean = jnp.mean(out, axis=0, keepdims=True)
    var = jnp.mean((out - mean) * (out - mean), axis=0, keepdims=True)
    bn = g_ref[...] * (out - mean) * lax.rsqrt(var + BN_EPS) + be_ref[...]
    o_ref[...] = jnp.maximum(bn, 0.0) + x_ref[...]


@jax.jit
def kernel(x, edge_index, W, b, gamma, beta):
    src = edge_index[0].astype(jnp.int32).reshape(NW, NCH, K)
    dst = edge_index[1].astype(jnp.int32).reshape(NW, NCH, K)
    x_pad = jnp.pad(x, ((0, NPAD - N_NODES), (0, 0)))
    zero_deg = jnp.zeros((NPAD,), jnp.float32)
    zero_rows = jnp.zeros((NPAD, DIM), jnp.float32)

    degp = _deg_kernel(dst, zero_deg)
    degt = degp.T

    y = pl.pallas_call(
        _prep_body,
        out_shape=jax.ShapeDtypeStruct((NPAD, DIM), jnp.float32),
    )(x_pad, W, degt)

    aggp = _agg_kernel(y, src, dst, zero_rows)

    out = pl.pallas_call(
        _post_body,
        out_shape=jax.ShapeDtypeStruct((N_NODES, DIM), jnp.float32),
    )(aggp, y, degt, x, b.reshape(1, DIM), gamma.reshape(1, DIM), beta.reshape(1, DIM))
    return out
